# Initial kernel scaffold; baseline (speedup 1.0000x reference)
import jax
import jax.numpy as jnp
from jax.experimental import pallas as pl


def kernel(z, edge_index, edge_type, rel_emb):
    raise NotImplementedError("write your pallas kernel here")


# speedup vs baseline: 2.7721x; 2.7721x over previous
import functools

import jax
import jax.numpy as jnp
from jax import lax
from jax.experimental import pallas as pl
from jax.experimental.pallas import tpu as pltpu
from jax.experimental.pallas import tpu_sc as plsc

_N_EDGES = 320000
_D = 128
_L = 16

_info = plsc.get_sparse_core_info()
_NC = _info.num_cores
_NS = _info.num_subcores
_NW = _NC * _NS
_EPW = _N_EDGES // _NW
_C = 80
_NCHUNK = _EPW // _C


_GATHER_DNUMS = lax.GatherDimensionNumbers(
    offset_dims=(), collapsed_slice_dims=(0,), start_index_map=(0,))


def _rot(x, idx):
    return lax.gather(x, idx[:, None], _GATHER_DNUMS, slice_sizes=(1,),
                      mode=lax.GatherScatterMode.PROMISE_IN_BOUNDS)


def _hsum_all_lanes(x):
    for k in (8, 4, 2, 1):
        idx = (lax.iota(jnp.int32, _L) + k) & (_L - 1)
        x = x + _rot(x, idx)
    return x


def _tec_body(z_hbm, src_hbm, dst_hbm, typ_hbm, rel_hbm, out_hbm,
              si, di, ti, sr, dr, rr, ob, sem):
    wid = lax.axis_index("s") * _NC + lax.axis_index("c")
    base = wid * _EPW

    def chunk(i, carry):
        off = base + i * _C
        pltpu.sync_copy(src_hbm.at[pl.ds(off, _C)], si)
        pltpu.sync_copy(dst_hbm.at[pl.ds(off, _C)], di)
        pltpu.sync_copy(typ_hbm.at[pl.ds(off, _C)], ti)
        c1 = pltpu.async_copy(z_hbm.at[si], sr, sem)
        c2 = pltpu.async_copy(z_hbm.at[di], dr, sem)
        c3 = pltpu.async_copy(rel_hbm.at[ti], rr, sem)
        c1.wait()
        c2.wait()
        c3.wait()

        def group(g, carry2):
            vec = jnp.zeros((_L,), jnp.float32)
            for l in range(_L):
                e = g * _L + l
                acc = jnp.zeros((_L,), jnp.float32)
                for j in range(_D // _L):
                    sl = pl.ds(j * _L, _L)
                    acc = acc + jnp.abs(sr[e, sl] + rr[e, sl] - dr[e, sl])
                lane = lax.iota(jnp.int32, _L) == l
                vec = jnp.where(lane, _hsum_all_lanes(acc), vec)
            ob[pl.ds(g * _L, _L)] = -vec
            return carry2

        lax.fori_loop(0, _C // _L, group, 0)
        pltpu.sync_copy(ob, out_hbm.at[pl.ds(off, _C)])
        return carry

    lax.fori_loop(0, _NCHUNK, chunk, 0)


_sc_call = pl.kernel(
    _tec_body,
    out_type=jax.ShapeDtypeStruct((_N_EDGES,), jnp.float32),
    mesh=plsc.VectorSubcoreMesh(core_axis_name="c", subcore_axis_name="s"),
    scratch_types=[
        pltpu.VMEM((_C,), jnp.int32),
        pltpu.VMEM((_C,), jnp.int32),
        pltpu.VMEM((_C,), jnp.int32),
        pltpu.VMEM((_C, _D), jnp.float32),
        pltpu.VMEM((_C, _D), jnp.float32),
        pltpu.VMEM((_C, _D), jnp.float32),
        pltpu.VMEM((_C,), jnp.float32),
        pltpu.SemaphoreType.DMA,
    ],
)


@jax.jit
def kernel(z, edge_index, edge_type, rel_emb):
    src = edge_index[0].astype(jnp.int32)
    dst = edge_index[1].astype(jnp.int32)
    typ = edge_type.astype(jnp.int32)
    return _sc_call(z, src, dst, typ, rel_emb)


# speedup vs baseline: 5.0671x; 1.8279x over previous
import functools

import jax
import jax.numpy as jnp
from jax import lax
from jax.experimental import pallas as pl
from jax.experimental.pallas import tpu as pltpu
from jax.experimental.pallas import tpu_sc as plsc

_N_EDGES = 320000
_D = 128
_L = 16

_info = plsc.get_sparse_core_info()
_NC = _info.num_cores
_NS = _info.num_subcores
_NW = _NC * _NS
_EPW = _N_EDGES // _NW
_C = 80
_NCHUNK = _EPW // _C

_GATHER_DNUMS = lax.GatherDimensionNumbers(
    offset_dims=(), collapsed_slice_dims=(0,), start_index_map=(0,))


def _rot(x, idx):
    return lax.gather(x, idx[:, None], _GATHER_DNUMS, slice_sizes=(1,),
                      mode=lax.GatherScatterMode.PROMISE_IN_BOUNDS)


def _hsum_all_lanes(x):
    for k in (8, 4, 2, 1):
        idx = (lax.iota(jnp.int32, _L) + k) & (_L - 1)
        x = x + _rot(x, idx)
    return x


def _tec_body(z_hbm, idx_hbm, rel_hbm, out_hbm,
              ib0, ib1, sr0, dr0, rr0, sr1, dr1, rr1, ob, sem0, sem1):
    wid = lax.axis_index("s") * _NC + lax.axis_index("c")
    base = wid * _EPW
    bufs = ((ib0, sr0, dr0, rr0, sem0), (ib1, sr1, dr1, rr1, sem1))

    def fire(c, buf):
        ib, sr, dr, rr, sem = buf
        goff = (wid * _NCHUNK + c) * 3 * _C
        pltpu.sync_copy(idx_hbm.at[pl.ds(goff, 3 * _C)], ib)
        pltpu.async_copy(z_hbm.at[ib.at[pl.ds(0, _C)]], sr, sem)
        pltpu.async_copy(z_hbm.at[ib.at[pl.ds(_C, _C)]], dr, sem)
        pltpu.async_copy(rel_hbm.at[ib.at[pl.ds(2 * _C, _C)]], rr, sem)

    def drain(buf):
        ib, sr, dr, rr, sem = buf
        pltpu.make_async_copy(z_hbm.at[ib.at[pl.ds(0, _C)]], sr, sem).wait()
        pltpu.make_async_copy(z_hbm.at[ib.at[pl.ds(_C, _C)]], dr, sem).wait()
        pltpu.make_async_copy(rel_hbm.at[ib.at[pl.ds(2 * _C, _C)]], rr, sem).wait()

    def compute(c, buf):
        _, sr, dr, rr, _ = buf
        off = base + c * _C

        def group(g, carry2):
            vec = jnp.zeros((_L,), jnp.float32)
            for l in range(_L):
                e = g * _L + l
                acc = jnp.zeros((_L,), jnp.float32)
                for j in range(_D // _L):
                    sl = pl.ds(j * _L, _L)
                    acc = acc + jnp.abs(sr[e, sl] + rr[e, sl] - dr[e, sl])
                lane = lax.iota(jnp.int32, _L) == l
                vec = jnp.where(lane, _hsum_all_lanes(acc), vec)
            ob[pl.ds(g * _L, _L)] = -vec
            return carry2

        lax.fori_loop(0, _C // _L, group, 0)
        pltpu.sync_copy(ob, out_hbm.at[pl.ds(off, _C)])

    fire(0, bufs[0])

    def pair(k, carry):
        c0 = k * 2
        fire(c0 + 1, bufs[1])
        drain(bufs[0])
        compute(c0, bufs[0])
        fire(c0 + 2, bufs[0])
        drain(bufs[1])
        compute(c0 + 1, bufs[1])
        return carry

    lax.fori_loop(0, (_NCHUNK - 1) // 2, pair, 0)
    drain(bufs[0])
    compute(_NCHUNK - 1, bufs[0])


_sc_call = pl.kernel(
    _tec_body,
    out_type=jax.ShapeDtypeStruct((_N_EDGES,), jnp.float32),
    mesh=plsc.VectorSubcoreMesh(core_axis_name="c", subcore_axis_name="s"),
    scratch_types=[
        pltpu.VMEM((3 * _C,), jnp.int32),
        pltpu.VMEM((3 * _C,), jnp.int32),
        pltpu.VMEM((_C, _D), jnp.float32),
        pltpu.VMEM((_C, _D), jnp.float32),
        pltpu.VMEM((_C, _D), jnp.float32),
        pltpu.VMEM((_C, _D), jnp.float32),
        pltpu.VMEM((_C, _D), jnp.float32),
        pltpu.VMEM((_C, _D), jnp.float32),
        pltpu.VMEM((_C,), jnp.float32),
        pltpu.SemaphoreType.DMA,
        pltpu.SemaphoreType.DMA,
    ],
)


@jax.jit
def kernel(z, edge_index, edge_type, rel_emb):
    idx_all = jnp.concatenate(
        [edge_index.astype(jnp.int32), edge_type.astype(jnp.int32)[None]], axis=0)
    idx_flat = jnp.transpose(
        idx_all.reshape(3, _N_EDGES // _C, _C), (1, 0, 2)).reshape(-1)
    return _sc_call(z, idx_flat, rel_emb)
